# baseline (speedup 1.0000x reference)
import jax
import jax.numpy as jnp
from jax.experimental import pallas as pl


BATCH_BLOCK = 32


def _add_body(x_ref, t_ref, o_ref):
    o_ref[...] = x_ref[...] + t_ref[...]


def kernel(x, pos_table):
    batch, seq_len, embed = x.shape
    flat = seq_len * embed
    x2 = x.reshape(batch, flat)
    t2 = pos_table[:seq_len].reshape(1, flat)
    grid = (batch // BATCH_BLOCK,)
    out = pl.pallas_call(
        _add_body,
        grid=grid,
        in_specs=[
            pl.BlockSpec((BATCH_BLOCK, flat), lambda i: (i, 0)),
            pl.BlockSpec((1, flat), lambda i: (0, 0)),
        ],
        out_specs=pl.BlockSpec((BATCH_BLOCK, flat), lambda i: (i, 0)),
        out_shape=jax.ShapeDtypeStruct((batch, flat), x.dtype),
    )(x2, t2)
    return out.reshape(batch, seq_len, embed)


# speedup vs baseline: 1.2983x; 1.2983x over previous
import jax
import jax.numpy as jnp
from jax.experimental import pallas as pl


BATCH_BLOCK = 32


def _add_body(x_ref, t_ref, o_ref):
    o_ref[...] = x_ref[...] + t_ref[...][None, :, :]


def kernel(x, pos_table):
    batch, seq_len, embed = x.shape
    table = pos_table[:seq_len]
    grid = (batch // BATCH_BLOCK,)
    return pl.pallas_call(
        _add_body,
        grid=grid,
        in_specs=[
            pl.BlockSpec((BATCH_BLOCK, seq_len, embed), lambda i: (i, 0, 0)),
            pl.BlockSpec((seq_len, embed), lambda i: (0, 0)),
        ],
        out_specs=pl.BlockSpec((BATCH_BLOCK, seq_len, embed), lambda i: (i, 0, 0)),
        out_shape=jax.ShapeDtypeStruct((batch, seq_len, embed), x.dtype),
    )(x, table)


# speedup vs baseline: 6.1762x; 4.7573x over previous
import jax
import jax.numpy as jnp
from jax.experimental import pallas as pl


BATCH_BLOCK = 32


def _add_body(x_ref, t_ref, o_ref):
    o_ref[...] = x_ref[...] + t_ref[...][None, :, :]


def kernel(x, pos_table):
    batch, seq_len, embed = x.shape
    xt = jnp.transpose(x, (0, 2, 1))
    tt = jnp.transpose(pos_table[:seq_len], (1, 0))
    grid = (batch // BATCH_BLOCK,)
    out_t = pl.pallas_call(
        _add_body,
        grid=grid,
        in_specs=[
            pl.BlockSpec((BATCH_BLOCK, embed, seq_len), lambda i: (i, 0, 0)),
            pl.BlockSpec((embed, seq_len), lambda i: (0, 0)),
        ],
        out_specs=pl.BlockSpec((BATCH_BLOCK, embed, seq_len), lambda i: (i, 0, 0)),
        out_shape=jax.ShapeDtypeStruct((batch, embed, seq_len), x.dtype),
    )(xt, tt)
    return jnp.transpose(out_t, (0, 2, 1))


# speedup vs baseline: 6.1997x; 1.0038x over previous
import jax
import jax.numpy as jnp
from jax.experimental import pallas as pl


BATCH_BLOCK = 64


def _add_body(x_ref, t_ref, o_ref):
    o_ref[...] = x_ref[...] + t_ref[...][None, :, :]


def kernel(x, pos_table):
    batch, seq_len, embed = x.shape
    xt = jnp.transpose(x, (0, 2, 1))
    tt = jnp.transpose(pos_table[:seq_len], (1, 0))
    grid = (batch // BATCH_BLOCK,)
    out_t = pl.pallas_call(
        _add_body,
        grid=grid,
        in_specs=[
            pl.BlockSpec((BATCH_BLOCK, embed, seq_len), lambda i: (i, 0, 0)),
            pl.BlockSpec((embed, seq_len), lambda i: (0, 0)),
        ],
        out_specs=pl.BlockSpec((BATCH_BLOCK, embed, seq_len), lambda i: (i, 0, 0)),
        out_shape=jax.ShapeDtypeStruct((batch, embed, seq_len), x.dtype),
    )(xt, tt)
    return jnp.transpose(out_t, (0, 2, 1))
